# baseline (speedup 1.0000x reference)
import functools

import numpy as np

import jax
import jax.numpy as jnp
from jax import lax
from jax.experimental import pallas as pl
from jax.experimental.pallas import tpu as pltpu

N = 4096
F = 4096
D = 64
BN = 256
BIG_I = 2**30
BIG_F = 3.0e38

def _top2_body(x_ref, c_ref, w_ref, i0_ref, i1_ref, wfb_ref):
    x = x_ref[...]
    c = c_ref[...]
    xx = jnp.sum(x * x, axis=1, keepdims=True)
    cc = jnp.sum(c * c, axis=1)
    xc = lax.dot_general(x, c, (((1,), (1,)), ((), ())),
                         preferred_element_type=jnp.float32)
    d2 = xx - 2.0 * xc + cc[None, :]
    iota = lax.broadcasted_iota(jnp.int32, (BN, F), 1)
    m1 = jnp.min(d2, axis=1)
    i1 = jnp.min(jnp.where(d2 <= m1[:, None], iota, BIG_I), axis=1)
    d2b = jnp.where(iota == i1[:, None], BIG_F, d2)
    m2 = jnp.min(d2b, axis=1)
    i2 = jnp.min(jnp.where(d2b <= m2[:, None], iota, BIG_I), axis=1)
    i0_ref[...] = i1
    i1_ref[...] = i2
    w = w_ref[...]
    ulo = lax.bitcast_convert_type(w[:, 0:32, :], jnp.uint32)
    uhi = lax.bitcast_convert_type(w[:, 32:64, :], jnp.uint32)
    lo16 = (ulo + 0x8000) >> 16
    hi16 = (uhi + 0x8000) & jnp.uint32(0xFFFF0000)
    packed = lax.bitcast_convert_type(lo16 | hi16, jnp.int32)
    wfb_ref[...] = jnp.concatenate(
        [packed[:, 0:16, :], packed[:, 16:32, :]], axis=2)


def _top2(x, ctrs, wts):
    grid = (N // BN,)
    return pl.pallas_call(
        _top2_body,
        grid=grid,
        in_specs=[
            pl.BlockSpec((BN, D), lambda i: (i, 0)),
            pl.BlockSpec((F, D), lambda i: (0, 0)),
            pl.BlockSpec((BN, D, D), lambda i: (i, 0, 0)),
        ],
        out_specs=[
            pl.BlockSpec((BN,), lambda i: (i,)),
            pl.BlockSpec((BN,), lambda i: (i,)),
            pl.BlockSpec((BN, 16, 128), lambda i: (i, 0, 0)),
        ],
        out_shape=[
            jax.ShapeDtypeStruct((N,), jnp.int32),
            jax.ShapeDtypeStruct((N,), jnp.int32),
            jax.ShapeDtypeStruct((F, 16, 128), jnp.int32),
        ],
    )(x, ctrs, wts)


NW = 32
SPW = N // NW
CH = 4
NCH = SPW // CH


def _sc_combine_body(x_hbm, co_hbm, wfb_hbm, idxp_hbm, y_hbm,
                     idxb0, idxb1, wb0, wb1, cob0, cob1, xbuf, ybuf,
                     sem0, sem1):
    from jax.experimental.pallas import tpu_sc as plsc
    wid = lax.axis_index("s") * 2 + lax.axis_index("c")
    base = wid * SPW
    pltpu.sync_copy(x_hbm.at[pl.ds(base, SPW)], xbuf)
    idxbs, wbs, cobs, sems = (idxb0, idxb1), (wb0, wb1), (cob0, cob1), (sem0, sem1)

    def fire(ci, b):
        pltpu.sync_copy(idxp_hbm.at[wid, ci], idxbs[b])
        pltpu.async_copy(wfb_hbm.at[idxbs[b]], wbs[b], sems[b])
        pltpu.async_copy(co_hbm.at[idxbs[b]], cobs[b], sems[b])

    def drain(b):
        pltpu.make_async_copy(wfb_hbm.at[idxbs[b]], wbs[b], sems[b]).wait()
        pltpu.make_async_copy(co_hbm.at[idxbs[b]], cobs[b], sems[b]).wait()

    def compute(ci, b):
        wb, cob = wbs[b], cobs[b]

        for s in range(CH):
            row = ci * CH + s
            acc = (jnp.zeros((16,), jnp.float32),) * 4
            for k in range(2):
                r = k * CH + s

                def dqbody(dq, a, r=r):
                    xchunk_lo = (xbuf[row, pl.ds(dq * 16, 16)]
                                 - cob[r, pl.ds(dq * 16, 16)])
                    xchunk_hi = (xbuf[row, pl.ds(32 + dq * 16, 16)]
                                 - cob[r, pl.ds(32 + dq * 16, 16)])
                    for j in range(16):
                        xlo = xchunk_lo[j]
                        xhi = xchunk_hi[j]
                        a0, a1, a2, a3 = a
                        u = j
                        vb = dq * 64
                        wua = wb[r, u, pl.ds(vb, 16)]
                        wub = wb[r, u, pl.ds(vb + 16, 16)]
                        wuc = wb[r, u, pl.ds(vb + 32, 16)]
                        wud = wb[r, u, pl.ds(vb + 48, 16)]
                        a0 = a0 + xlo * plsc.bitcast(wua << 16, jnp.float32)
                        a1 = a1 + xlo * plsc.bitcast(wub << 16, jnp.float32)
                        a2 = a2 + xlo * plsc.bitcast(wuc << 16, jnp.float32)
                        a3 = a3 + xlo * plsc.bitcast(wud << 16, jnp.float32)
                        a0 = a0 + xhi * plsc.bitcast(wua, jnp.float32)
                        a1 = a1 + xhi * plsc.bitcast(wub, jnp.float32)
                        a2 = a2 + xhi * plsc.bitcast(wuc, jnp.float32)
                        a3 = a3 + xhi * plsc.bitcast(wud, jnp.float32)
                        a = (a0, a1, a2, a3)
                    return a

                acc = lax.fori_loop(0, 2, dqbody, acc)
            for q in range(4):
                ybuf[row, pl.ds(q * 16, 16)] = (
                    acc[q] + cob[s, pl.ds(D + q * 16, 16)]
                    + cob[CH + s, pl.ds(D + q * 16, 16)])

    fire(0, 0)

    def gbody(g, carry):
        ci1 = 2 * g + 1
        fire(ci1, 1)
        drain(0)
        compute(2 * g, 0)

        @pl.when(g < NCH // 2 - 1)
        def _():
            fire(ci1 + 1, 0)

        drain(1)
        compute(ci1, 1)
        return carry

    lax.fori_loop(0, NCH // 2, gbody, 0)
    pltpu.sync_copy(ybuf, y_hbm.at[pl.ds(base, SPW)])


def _sc_combine(x, co, wfb, idxp):
    from jax.experimental.pallas import tpu_sc as plsc
    mesh = plsc.VectorSubcoreMesh(
        core_axis_name="c", subcore_axis_name="s", num_cores=2, num_subcores=16)
    return pl.kernel(
        _sc_combine_body,
        out_type=jax.ShapeDtypeStruct((N, D), jnp.float32),
        mesh=mesh,
        compiler_params=pltpu.CompilerParams(needs_layout_passes=False),
        scratch_types=[
            pltpu.VMEM((2 * CH,), jnp.int32),
            pltpu.VMEM((2 * CH,), jnp.int32),
            pltpu.VMEM((2 * CH, 16, 128), jnp.int32),
            pltpu.VMEM((2 * CH, 16, 128), jnp.int32),
            pltpu.VMEM((2 * CH, 2 * D), jnp.float32),
            pltpu.VMEM((2 * CH, 2 * D), jnp.float32),
            pltpu.VMEM((SPW, D), jnp.float32),
            pltpu.VMEM((SPW, D), jnp.float32),
            pltpu.SemaphoreType.DMA,
            pltpu.SemaphoreType.DMA,
        ],
    )(x, co, wfb, idxp)


def kernel(x, ctrs, wts, offsets):
    i0, i1, wfb = _top2(x, ctrs, wts)
    idxp = (jnp.stack([i0, i1])
            .reshape(2, NW, NCH, CH)
            .transpose(1, 2, 0, 3)
            .reshape(NW, NCH, 2 * CH))
    co = jnp.concatenate([ctrs, offsets], axis=1)
    return _sc_combine(x, co, wfb, idxp)


# speedup vs baseline: 1.0561x; 1.0561x over previous
import functools

import numpy as np

import jax
import jax.numpy as jnp
from jax import lax
from jax.experimental import pallas as pl
from jax.experimental.pallas import tpu as pltpu

N = 4096
F = 4096
D = 64
BN = 256
BIG_I = 2**30
BIG_F = 3.0e38

def _top2_body(x_ref, c_ref, i0_ref, i1_ref):
    x = x_ref[...]
    c = c_ref[...]
    xx = jnp.sum(x * x, axis=1, keepdims=True)
    cc = jnp.sum(c * c, axis=1)
    xc = lax.dot_general(x, c, (((1,), (1,)), ((), ())),
                         preferred_element_type=jnp.float32)
    d2 = xx - 2.0 * xc + cc[None, :]
    iota = lax.broadcasted_iota(jnp.int32, (BN, F), 1)
    m1 = jnp.min(d2, axis=1)
    i1 = jnp.min(jnp.where(d2 <= m1[:, None], iota, BIG_I), axis=1)
    d2b = jnp.where(iota == i1[:, None], BIG_F, d2)
    m2 = jnp.min(d2b, axis=1)
    i2 = jnp.min(jnp.where(d2b <= m2[:, None], iota, BIG_I), axis=1)
    i0_ref[...] = i1
    i1_ref[...] = i2


def _top2(x, ctrs):
    grid = (N // BN,)
    return pl.pallas_call(
        _top2_body,
        grid=grid,
        in_specs=[
            pl.BlockSpec((BN, D), lambda i: (i, 0)),
            pl.BlockSpec((F, D), lambda i: (0, 0)),
        ],
        out_specs=[
            pl.BlockSpec((BN,), lambda i: (i,)),
            pl.BlockSpec((BN,), lambda i: (i,)),
        ],
        out_shape=[
            jax.ShapeDtypeStruct((N,), jnp.int32),
            jax.ShapeDtypeStruct((N,), jnp.int32),
        ],
    )(x, ctrs)


NW = 32
SPW = N // NW
CH = 4
NCH = SPW // CH


def _sc_combine_body(x_hbm, co_hbm, wfb_hbm, idxp_hbm, y_hbm,
                     idxb0, idxb1, wb0, wb1, cob0, cob1, xbuf, ybuf, xts,
                     sem0, sem1):
    from jax.experimental.pallas import tpu_sc as plsc
    wid = lax.axis_index("s") * 2 + lax.axis_index("c")
    base = wid * SPW
    pltpu.sync_copy(x_hbm.at[pl.ds(base, SPW)], xbuf)
    idxbs, wbs, cobs, sems = (idxb0, idxb1), (wb0, wb1), (cob0, cob1), (sem0, sem1)

    def fire(ci, b):
        pltpu.sync_copy(idxp_hbm.at[wid, ci], idxbs[b])
        pltpu.async_copy(wfb_hbm.at[idxbs[b]], wbs[b], sems[b])
        pltpu.async_copy(co_hbm.at[idxbs[b]], cobs[b], sems[b])

    def drain(b):
        pltpu.make_async_copy(wfb_hbm.at[idxbs[b]], wbs[b], sems[b]).wait()
        pltpu.make_async_copy(co_hbm.at[idxbs[b]], cobs[b], sems[b]).wait()

    def compute(ci, b):
        wb, cob = wbs[b], cobs[b]

        for s in range(CH):
            row = ci * CH + s
            acc = (jnp.zeros((16,), jnp.float32),) * 4
            for k in range(2):
                r = k * CH + s
                for q in range(4):
                    xts[pl.ds(q * 16, 16)] = (
                        xbuf[row, pl.ds(q * 16, 16)] - cob[r, pl.ds(q * 16, 16)])

                def dqbody(dq, a, r=r):
                    ivlo = jnp.full((16,), dq * 16, jnp.int32)
                    for j in range(16):
                        a0, a1, a2, a3 = a
                        xlo = plsc.load_gather(xts, [ivlo + j])
                        xhi = plsc.load_gather(xts, [ivlo + (32 + j)])
                        vb = dq * 64
                        wua = wb[r, j, pl.ds(vb, 16)]
                        wub = wb[r, j, pl.ds(vb + 16, 16)]
                        wuc = wb[r, j, pl.ds(vb + 32, 16)]
                        wud = wb[r, j, pl.ds(vb + 48, 16)]
                        a0 = a0 + xlo * plsc.bitcast(wua << 16, jnp.float32)
                        a1 = a1 + xlo * plsc.bitcast(wub << 16, jnp.float32)
                        a2 = a2 + xlo * plsc.bitcast(wuc << 16, jnp.float32)
                        a3 = a3 + xlo * plsc.bitcast(wud << 16, jnp.float32)
                        a0 = a0 + xhi * plsc.bitcast(wua, jnp.float32)
                        a1 = a1 + xhi * plsc.bitcast(wub, jnp.float32)
                        a2 = a2 + xhi * plsc.bitcast(wuc, jnp.float32)
                        a3 = a3 + xhi * plsc.bitcast(wud, jnp.float32)
                        a = (a0, a1, a2, a3)
                    return a

                acc = lax.fori_loop(0, 2, dqbody, acc)
            for q in range(4):
                ybuf[row, pl.ds(q * 16, 16)] = (
                    acc[q] + cob[s, pl.ds(D + q * 16, 16)]
                    + cob[CH + s, pl.ds(D + q * 16, 16)])

    fire(0, 0)

    def gbody(g, carry):
        ci1 = 2 * g + 1
        fire(ci1, 1)
        drain(0)
        compute(2 * g, 0)

        @pl.when(g < NCH // 2 - 1)
        def _():
            fire(ci1 + 1, 0)

        drain(1)
        compute(ci1, 1)
        return carry

    lax.fori_loop(0, NCH // 2, gbody, 0)
    pltpu.sync_copy(ybuf, y_hbm.at[pl.ds(base, SPW)])


def _sc_combine(x, co, wfb, idxp):
    from jax.experimental.pallas import tpu_sc as plsc
    mesh = plsc.VectorSubcoreMesh(
        core_axis_name="c", subcore_axis_name="s", num_cores=2, num_subcores=16)
    return pl.kernel(
        _sc_combine_body,
        out_type=jax.ShapeDtypeStruct((N, D), jnp.float32),
        mesh=mesh,
        compiler_params=pltpu.CompilerParams(needs_layout_passes=False),
        scratch_types=[
            pltpu.VMEM((2 * CH,), jnp.int32),
            pltpu.VMEM((2 * CH,), jnp.int32),
            pltpu.VMEM((2 * CH, 16, 128), jnp.int32),
            pltpu.VMEM((2 * CH, 16, 128), jnp.int32),
            pltpu.VMEM((2 * CH, 2 * D), jnp.float32),
            pltpu.VMEM((2 * CH, 2 * D), jnp.float32),
            pltpu.VMEM((SPW, D), jnp.float32),
            pltpu.VMEM((SPW, D), jnp.float32),
            pltpu.VMEM((D,), jnp.float32),
            pltpu.SemaphoreType.DMA,
            pltpu.SemaphoreType.DMA,
        ],
    )(x, co, wfb, idxp)


def _pack_wts(wts):
    u32 = lax.bitcast_convert_type(wts, jnp.uint32)
    lo16 = (u32[:, 0:32, :] + 0x8000) >> 16
    hi16 = (u32[:, 32:64, :] + 0x8000) & jnp.uint32(0xFFFF0000)
    p = lax.bitcast_convert_type(lo16 | hi16, jnp.int32)
    return (p.reshape(F, 2, 16, D)
            .transpose(0, 2, 1, 3)
            .reshape(F, 16, 2 * D))


def kernel(x, ctrs, wts, offsets):
    i0, i1 = _top2(x, ctrs)
    wfb = _pack_wts(wts)
    idxp = (jnp.stack([i0, i1])
            .reshape(2, NW, NCH, CH)
            .transpose(1, 2, 0, 3)
            .reshape(NW, NCH, 2 * CH))
    co = jnp.concatenate([ctrs, offsets], axis=1)
    return _sc_combine(x, co, wfb, idxp)


# speedup vs baseline: 1.4782x; 1.3996x over previous
import functools

import jax
import jax.numpy as jnp
from jax import lax
from jax.experimental import pallas as pl
from jax.experimental.pallas import tpu as pltpu

N = 4096
F = 4096
D = 64
BN = 256
BIG_I = 2**30
BIG_F = 3.0e38


def _top2_body(x_ref, c_ref, i0_ref, i1_ref):
    x = x_ref[...]
    c = c_ref[...]
    xx = jnp.sum(x * x, axis=1, keepdims=True)
    cc = jnp.sum(c * c, axis=1)
    xc = lax.dot_general(x, c, (((1,), (1,)), ((), ())),
                         preferred_element_type=jnp.float32)
    d2 = xx - 2.0 * xc + cc[None, :]
    iota = lax.broadcasted_iota(jnp.int32, (BN, F), 1)
    i1 = jnp.argmin(d2, axis=1).astype(jnp.int32)
    d2b = jnp.where(iota == i1[:, None], BIG_F, d2)
    i2 = jnp.argmin(d2b, axis=1).astype(jnp.int32)
    i0_ref[...] = i1
    i1_ref[...] = i2


def _top2(x, ctrs):
    grid = (N // BN,)
    return pl.pallas_call(
        _top2_body,
        grid=grid,
        in_specs=[
            pl.BlockSpec((BN, D), lambda i: (i, 0)),
            pl.BlockSpec((F, D), lambda i: (0, 0)),
        ],
        out_specs=[
            pl.BlockSpec((BN,), lambda i: (i,)),
            pl.BlockSpec((BN,), lambda i: (i,)),
        ],
        out_shape=[
            jax.ShapeDtypeStruct((N,), jnp.int32),
            jax.ShapeDtypeStruct((N,), jnp.int32),
        ],
    )(x, ctrs)


NW = 32
SPW = N // NW
CH = 4
NCH = SPW // CH


def _sc_combine_body(x_hbm, co_hbm, wflat_hbm, idxp_hbm, y_hbm,
                     idxb0, idxb1, wb0, wb1, cob0, cob1, xbuf, ybuf,
                     sem0, sem1):
    wid = lax.axis_index("s") * 2 + lax.axis_index("c")
    base = wid * SPW
    pltpu.sync_copy(x_hbm.at[pl.ds(base, SPW)], xbuf)
    idxbs, wbs, cobs, sems = (idxb0, idxb1), (wb0, wb1), (cob0, cob1), (sem0, sem1)

    def fire(ci, b):
        pltpu.sync_copy(idxp_hbm.at[wid, ci], idxbs[b])
        pltpu.async_copy(wflat_hbm.at[idxbs[b]], wbs[b], sems[b])
        pltpu.async_copy(co_hbm.at[idxbs[b]], cobs[b], sems[b])

    def drain(b):
        pltpu.make_async_copy(wflat_hbm.at[idxbs[b]], wbs[b], sems[b]).wait()
        pltpu.make_async_copy(co_hbm.at[idxbs[b]], cobs[b], sems[b]).wait()

    def compute(ci, b):
        wb, cob = wbs[b], cobs[b]

        def sbody(s, carry):
            row = ci * CH + s
            acc = (jnp.zeros((16,), jnp.float32),) * 4
            for k in range(2):
                r = k * CH + s

                def dqbody(dq, a):
                    xchunk = (xbuf[row, pl.ds(dq * 16, 16)]
                              - cob[r, pl.ds(dq * 16, 16)])
                    for j in range(16):
                        xv = xchunk[j]
                        a = tuple(
                            a[q] + xv * wb[r, pl.ds(dq * 1024 + j * 64 + q * 16, 16)]
                            for q in range(4))
                    return a

                acc = lax.fori_loop(0, 4, dqbody, acc)
            for q in range(4):
                ybuf[row, pl.ds(q * 16, 16)] = (
                    acc[q] + cob[s, pl.ds(D + q * 16, 16)]
                    + cob[CH + s, pl.ds(D + q * 16, 16)])
            return carry

        lax.fori_loop(0, CH, sbody, 0)

    fire(0, 0)

    def gbody(g, carry):
        ci1 = 2 * g + 1
        fire(ci1, 1)
        drain(0)
        compute(2 * g, 0)

        @pl.when(g < NCH // 2 - 1)
        def _():
            fire(ci1 + 1, 0)

        drain(1)
        compute(ci1, 1)
        return carry

    lax.fori_loop(0, NCH // 2, gbody, 0)
    pltpu.sync_copy(ybuf, y_hbm.at[pl.ds(base, SPW)])


def _sc_combine(x, co, wflat, idxp):
    from jax.experimental.pallas import tpu_sc as plsc
    mesh = plsc.VectorSubcoreMesh(
        core_axis_name="c", subcore_axis_name="s", num_cores=2, num_subcores=16)
    return pl.kernel(
        _sc_combine_body,
        out_type=jax.ShapeDtypeStruct((N, D), jnp.float32),
        mesh=mesh,
        scratch_types=[
            pltpu.VMEM((2 * CH,), jnp.int32),
            pltpu.VMEM((2 * CH,), jnp.int32),
            pltpu.VMEM((2 * CH, D * D), jnp.float32),
            pltpu.VMEM((2 * CH, D * D), jnp.float32),
            pltpu.VMEM((2 * CH, 2 * D), jnp.float32),
            pltpu.VMEM((2 * CH, 2 * D), jnp.float32),
            pltpu.VMEM((SPW, D), jnp.float32),
            pltpu.VMEM((SPW, D), jnp.float32),
            pltpu.SemaphoreType.DMA,
            pltpu.SemaphoreType.DMA,
        ],
    )(x, co, wflat, idxp)


def kernel(x, ctrs, wts, offsets):
    i0, i1 = _top2(x, ctrs)
    idxp = (jnp.stack([i0, i1])
            .reshape(2, NW, NCH, CH)
            .transpose(1, 2, 0, 3)
            .reshape(NW, NCH, 2 * CH))
    co = jnp.concatenate([ctrs, offsets], axis=1)
    wflat = wts.reshape(F, D * D)
    return _sc_combine(x, co, wflat, idxp)


# speedup vs baseline: 1.5681x; 1.0608x over previous
import functools

import jax
import jax.numpy as jnp
from jax import lax
from jax.experimental import pallas as pl
from jax.experimental.pallas import tpu as pltpu

N = 4096
F = 4096
D = 64
BN = 256
BIG_I = 2**30
BIG_F = 3.0e38


def _top2_body(x_ref, c_ref, i0_ref, i1_ref):
    x = x_ref[...]
    c = c_ref[...]
    xx = jnp.sum(x * x, axis=1, keepdims=True)
    cc = jnp.sum(c * c, axis=1)
    xc = lax.dot_general(x, c, (((1,), (1,)), ((), ())),
                         preferred_element_type=jnp.float32)
    d2 = xx - 2.0 * xc + cc[None, :]
    iota = lax.broadcasted_iota(jnp.int32, (BN, F), 1)
    i1 = jnp.argmin(d2, axis=1).astype(jnp.int32)
    d2b = jnp.where(iota == i1[:, None], BIG_F, d2)
    i2 = jnp.argmin(d2b, axis=1).astype(jnp.int32)
    i0_ref[...] = i1
    i1_ref[...] = i2


def _top2(x, ctrs):
    n = x.shape[0]
    grid = (n // BN,)
    return pl.pallas_call(
        _top2_body,
        grid=grid,
        in_specs=[
            pl.BlockSpec((BN, D), lambda i: (i, 0)),
            pl.BlockSpec((F, D), lambda i: (0, 0)),
        ],
        out_specs=[
            pl.BlockSpec((BN,), lambda i: (i,)),
            pl.BlockSpec((BN,), lambda i: (i,)),
        ],
        out_shape=[
            jax.ShapeDtypeStruct((n,), jnp.int32),
            jax.ShapeDtypeStruct((n,), jnp.int32),
        ],
    )(x, ctrs)


NW = 32
SPW = N // NW
CH = 4
NCH = SPW // CH


def _sc_combine_body(x_hbm, co_hbm, wflat_hbm, idxp_hbm, y_hbm,
                     idxb0, idxb1, wb0, wb1, cob0, cob1, xbuf, ybuf,
                     sem0, sem1, *, spw, nch):
    wid = lax.axis_index("s") * 2 + lax.axis_index("c")
    base = wid * spw
    pltpu.sync_copy(x_hbm.at[pl.ds(base, spw)], xbuf)
    idxbs, wbs, cobs, sems = (idxb0, idxb1), (wb0, wb1), (cob0, cob1), (sem0, sem1)

    def fire(ci, b):
        pltpu.sync_copy(idxp_hbm.at[wid, ci], idxbs[b])
        pltpu.async_copy(wflat_hbm.at[idxbs[b]], wbs[b], sems[b])
        pltpu.async_copy(co_hbm.at[idxbs[b]], cobs[b], sems[b])

    def drain(b):
        pltpu.make_async_copy(wflat_hbm.at[idxbs[b]], wbs[b], sems[b]).wait()
        pltpu.make_async_copy(co_hbm.at[idxbs[b]], cobs[b], sems[b]).wait()

    def compute(ci, b):
        wb, cob = wbs[b], cobs[b]

        def sbody(s, carry):
            row = ci * CH + s
            acc = (jnp.zeros((16,), jnp.float32),) * 4
            for k in range(2):
                r = k * CH + s

                def dqbody(dq, a):
                    xchunk = (xbuf[row, pl.ds(dq * 16, 16)]
                              - cob[r, pl.ds(dq * 16, 16)])
                    for j in range(16):
                        xv = xchunk[j]
                        a = tuple(
                            a[q] + xv * wb[r, pl.ds(dq * 1024 + j * 64 + q * 16, 16)]
                            for q in range(4))
                    return a

                acc = lax.fori_loop(0, 4, dqbody, acc)
            for q in range(4):
                ybuf[row, pl.ds(q * 16, 16)] = (
                    acc[q] + cob[s, pl.ds(D + q * 16, 16)]
                    + cob[CH + s, pl.ds(D + q * 16, 16)])
            return carry

        lax.fori_loop(0, CH, sbody, 0)

    fire(0, 0)

    def gbody(g, carry):
        ci1 = 2 * g + 1
        fire(ci1, 1)
        drain(0)
        compute(2 * g, 0)

        @pl.when(g < nch // 2 - 1)
        def _():
            fire(ci1 + 1, 0)

        drain(1)
        compute(ci1, 1)
        return carry

    lax.fori_loop(0, nch // 2, gbody, 0)
    pltpu.sync_copy(ybuf, y_hbm.at[pl.ds(base, spw)])


def _sc_combine(x, co, wflat, idxp):
    from jax.experimental.pallas import tpu_sc as plsc
    nh = x.shape[0]
    spw = nh // NW
    nch = spw // CH
    mesh = plsc.VectorSubcoreMesh(
        core_axis_name="c", subcore_axis_name="s", num_cores=2, num_subcores=16)
    return pl.kernel(
        functools.partial(_sc_combine_body, spw=spw, nch=nch),
        out_type=jax.ShapeDtypeStruct((nh, D), jnp.float32),
        mesh=mesh,
        scratch_types=[
            pltpu.VMEM((2 * CH,), jnp.int32),
            pltpu.VMEM((2 * CH,), jnp.int32),
            pltpu.VMEM((2 * CH, D * D), jnp.float32),
            pltpu.VMEM((2 * CH, D * D), jnp.float32),
            pltpu.VMEM((2 * CH, 2 * D), jnp.float32),
            pltpu.VMEM((2 * CH, 2 * D), jnp.float32),
            pltpu.VMEM((spw, D), jnp.float32),
            pltpu.VMEM((spw, D), jnp.float32),
            pltpu.SemaphoreType.DMA,
            pltpu.SemaphoreType.DMA,
        ],
    )(x, co, wflat, idxp)


def kernel(x, ctrs, wts, offsets):
    co = jnp.concatenate([ctrs, offsets], axis=1)
    wflat = wts.reshape(F, D * D)
    nh = N // 2
    ys = []
    for h in range(2):
        xh = lax.slice_in_dim(x, h * nh, (h + 1) * nh, axis=0)
        i0, i1 = _top2(xh, ctrs)
        nch = nh // NW // CH
        idxp = (jnp.stack([i0, i1])
                .reshape(2, NW, nch, CH)
                .transpose(1, 2, 0, 3)
                .reshape(NW, nch, 2 * CH))
        ys.append(_sc_combine(xh, co, wflat, idxp))
    return jnp.concatenate(ys, axis=0)
